# baseline (speedup 1.0000x reference)
import jax
import jax.numpy as jnp
from jax.experimental import pallas as pl
from jax.experimental.pallas import tpu as pltpu


def _body(lg_ref, bx_ref, ci_ref, lab_ref, box_ref):
    x = lg_ref[...]
    m = jnp.max(x, axis=-1, keepdims=True)
    eqh = (x == m).astype(jnp.bfloat16)
    a = jax.lax.dot_general(
        eqh, ci_ref[...], (((1,), (0,)), ((), ())),
        preferred_element_type=jnp.float32,
    )
    valid = m > 0.0
    lab_ref[...] = jnp.where(valid, a.astype(jnp.int32), -1)
    box_ref[...] = jnp.where(valid, bx_ref[...], 0.0)


def kernel(pred_logits, pred_boxes):
    B, Q, C = pred_logits.shape
    R = B * Q
    lg = pred_logits.reshape(R, C)
    bx = pred_boxes.reshape(R, 4)
    cidx = jnp.arange(C, dtype=jnp.int32).astype(jnp.bfloat16).reshape(C, 1)

    BR = 4096
    labels, boxes = pl.pallas_call(
        _body,
        grid=(R // BR,),
        in_specs=[
            pl.BlockSpec((BR, C), lambda i: (i, 0)),
            pl.BlockSpec((BR, 4), lambda i: (i, 0)),
            pl.BlockSpec((C, 1), lambda i: (0, 0)),
        ],
        out_specs=[
            pl.BlockSpec((BR, 1), lambda i: (i, 0)),
            pl.BlockSpec((BR, 4), lambda i: (i, 0)),
        ],
        out_shape=[
            jax.ShapeDtypeStruct((R, 1), jnp.int32),
            jax.ShapeDtypeStruct((R, 4), jnp.float32),
        ],
    )(lg, bx, cidx)
    labels = labels[:, 0]
    num_boxes = jnp.maximum(jnp.sum(labels >= 0).astype(jnp.float32), 1.0)
    return labels.reshape(B, Q), boxes.reshape(B, Q, 4), num_boxes
